# baseline (speedup 1.0000x reference)
import functools

import jax
import jax.numpy as jnp
from jax import lax
from jax.experimental import pallas as pl
from jax.experimental.pallas import tpu as pltpu
from jax.experimental.pallas import tpu_sc as plsc

_N = 10000
_E = 320000
_D = 128
_BN_EPS = 1e-5

_CH = 128


def _sc_agg(table, src_r, dst_r, zeros, *, nc, ns, nch0, nch1, acc_rows):
    n, d = table.shape
    nch = max(nch0, nch1)
    assert nch0 % 8 == 0 or True
    rows_per_tile_z = acc_rows // ns
    rows_cp = (n // ns) // 8 * 8
    tail = n - rows_cp * ns
    mesh = plsc.VectorSubcoreMesh(core_axis_name="c", subcore_axis_name="s")

    @functools.partial(
        pl.kernel,
        out_type=jax.ShapeDtypeStruct((nc, n, d), jnp.float32),
        mesh=mesh,
        scratch_types=[
            pltpu.VMEM((nch, _CH), jnp.int32),
            pltpu.VMEM((nch, _CH), jnp.int32),
            pltpu.VMEM((_CH, d), jnp.float32),
            pltpu.VMEM_SHARED((acc_rows, d), jnp.float32),
            pltpu.SemaphoreType.DMA,
        ],
    )
    def body(table_hbm, src0_hbm, dst0_hbm, src1_hbm, dst1_hbm, z_hbm,
             out_hbm, idx_s, idx_d, rows, acc, sem):
        cid = lax.axis_index("c")
        sid = lax.axis_index("s")
        my_nch = jnp.where(cid == 0, nch0, nch1)

        pltpu.sync_copy(z_hbm, acc.at[pl.ds(sid * rows_per_tile_z,
                                            rows_per_tile_z)])

        @pl.when(cid == 0)
        def _():
            pltpu.sync_copy(src0_hbm.at[sid], idx_s.at[pl.ds(0, nch0)])
            pltpu.sync_copy(dst0_hbm.at[sid], idx_d.at[pl.ds(0, nch0)])

        @pl.when(cid != 0)
        def _():
            pltpu.sync_copy(src1_hbm.at[sid], idx_s.at[pl.ds(0, nch1)])
            pltpu.sync_copy(dst1_hbm.at[sid], idx_d.at[pl.ds(0, nch1)])
        plsc.subcore_barrier()

        def chunk(c, carry):
            pltpu.async_copy(table_hbm.at[idx_s.at[c]], rows, sem).wait()
            pltpu.sync_copy(rows, acc.at[idx_d.at[c]], add=True)
            return carry

        lax.fori_loop(0, my_nch, chunk, 0, unroll=False)
        plsc.subcore_barrier()

        pltpu.sync_copy(
            acc.at[pl.ds(sid * rows_cp, rows_cp)],
            out_hbm.at[cid].at[pl.ds(sid * rows_cp, rows_cp)],
        )
        if tail:
            @pl.when(sid == ns - 1)
            def _():
                pltpu.sync_copy(
                    acc.at[pl.ds(rows_cp * ns, tail)],
                    out_hbm.at[cid].at[pl.ds(rows_cp * ns, tail)],
                )

    return body(table, src_r[0], dst_r[0], src_r[1], dst_r[1], zeros)


def _mlp_block(x_ref, p_ref, wa_ref, ca_ref, wb_ref, cb_ref):
    a = x_ref[...] + p_ref[0] + p_ref[1]
    t = jnp.dot(a, wa_ref[...], preferred_element_type=jnp.float32)
    t = jnp.maximum(t + ca_ref[...], 0.0)
    u = jnp.dot(t, wb_ref[...], preferred_element_type=jnp.float32)
    return u + cb_ref[...]


def _mlp1_body(x_ref, p_ref, wa_ref, ca_ref, wb_ref, cb_ref, o_ref):
    u = _mlp_block(x_ref, p_ref, wa_ref, ca_ref, wb_ref, cb_ref)
    o_ref[...] = jnp.maximum(u, 0.0)


def _mlp2_body(x_ref, p_ref, wa_ref, ca_ref, wb_ref, cb_ref, o_ref):
    u = _mlp_block(x_ref, p_ref, wa_ref, ca_ref, wb_ref, cb_ref)
    m = jnp.max(u, axis=1, keepdims=True)
    z = u - m
    lse = jnp.log(jnp.sum(jnp.exp(z), axis=1, keepdims=True))
    o_ref[...] = z - lse


def _tc_mlp(body_fn, x, p, wa, ca, wb, cb, *, block_rows=1000):
    n, d = x.shape
    grid = (n // block_rows,)
    return pl.pallas_call(
        body_fn,
        grid=grid,
        in_specs=[
            pl.BlockSpec((block_rows, d), lambda i: (i, 0)),
            pl.BlockSpec((p.shape[0], block_rows, d), lambda i: (0, i, 0)),
            pl.BlockSpec((d, d), lambda i: (0, 0)),
            pl.BlockSpec((1, d), lambda i: (0, 0)),
            pl.BlockSpec((d, d), lambda i: (0, 0)),
            pl.BlockSpec((1, d), lambda i: (0, 0)),
        ],
        out_specs=pl.BlockSpec((block_rows, d), lambda i: (i, 0)),
        out_shape=jax.ShapeDtypeStruct((n, d), jnp.float32),
    )(x, p, wa, ca, wb, cb)


def kernel(x, edge_index, W1, b1, g1, bt1, W2, b2, bn_g0, bn_b0,
           W3, b3, g2, bt2, W4, b4):
    info = plsc.get_sparse_core_info()
    nc, ns = info.num_cores, info.num_subcores
    nw = nc * ns

    inv = 1.0 / jnp.sqrt(1.0 + _BN_EPS)
    s1 = g1 * inv
    w1f = W1.T * s1[None, :]
    c1 = (b1 * s1 + bt1)[None, :]
    s0 = bn_g0 * inv
    w2f = W2.T * s0[None, :]
    c2 = (b2 * s0 + bn_b0)[None, :]
    s2 = g2 * inv
    w3f = W3.T * s2[None, :]
    c3 = (b3 * s2 + bt2)[None, :]
    w4f = W4.T
    c4 = b4[None, :]

    zr = (-(-(_N + 1) // ns) + 7) // 8 * 8
    acc_rows = zr * ns
    zeros = jnp.zeros((zr, _D), jnp.float32)

    e = edge_index.shape[1]
    nchunks = -(-e // _CH)
    nch0 = -(-(nchunks * 11) // (16 * ns))
    nch1 = -(-(nchunks - nch0 * ns) // ns)
    cap = ns * (nch0 + nch1) * _CH
    npad = cap - e
    src = edge_index[0]
    dst = edge_index[1]
    pad_dst = _N + (jnp.arange(npad, dtype=jnp.int32) % (acc_rows - _N))
    src_p = jnp.concatenate([src, jnp.zeros((npad,), jnp.int32)])
    dst_p = jnp.concatenate([dst, pad_dst])

    def layout(arr):
        cut = ns * nch0 * _CH
        return (arr[:cut].reshape(ns, nch0, _CH),
                arr[cut:].reshape(ns, nch1, _CH))

    src_r = layout(src_p)
    dst_r = layout(dst_p)

    agg = functools.partial(_sc_agg, src_r=src_r, dst_r=dst_r, zeros=zeros,
                            nc=nc, ns=ns, nch0=nch0, nch1=nch1,
                            acc_rows=acc_rows)

    p1 = agg(x)
    h = _tc_mlp(_mlp1_body, x, p1, w1f, c1, w2f, c2)
    p2 = agg(h)
    return _tc_mlp(_mlp2_body, h, p2, w3f, c3, w4f, c4)


# speedup vs baseline: 1.0740x; 1.0740x over previous
import functools

import jax
import jax.numpy as jnp
from jax import lax
from jax.experimental import pallas as pl
from jax.experimental.pallas import tpu as pltpu
from jax.experimental.pallas import tpu_sc as plsc

_N = 10000
_E = 320000
_D = 128
_BN_EPS = 1e-5

_CH = 128


def _sc_agg(table, src_r, dst_r, zeros, *, nc, ns, nch0, nch1, acc_rows):
    n, d = table.shape
    nch = max(nch0, nch1)
    assert nch0 % 8 == 0 or True
    rows_per_tile_z = acc_rows // ns
    rows_cp = (n // ns) // 8 * 8
    tail = n - rows_cp * ns
    mesh = plsc.VectorSubcoreMesh(core_axis_name="c", subcore_axis_name="s")

    @functools.partial(
        pl.kernel,
        out_type=jax.ShapeDtypeStruct((nc, n, d), jnp.float32),
        mesh=mesh,
        scratch_types=[
            pltpu.VMEM((nch, _CH), jnp.int32),
            pltpu.VMEM((nch, _CH), jnp.int32),
            pltpu.VMEM((_CH, d), jnp.float32),
            pltpu.VMEM_SHARED((acc_rows, d), jnp.float32),
            pltpu.SemaphoreType.DMA,
        ],
    )
    def body(table_hbm, src0_hbm, dst0_hbm, src1_hbm, dst1_hbm, z_hbm,
             out_hbm, idx_s, idx_d, rows, acc, sem):
        cid = lax.axis_index("c")
        sid = lax.axis_index("s")
        my_nch = jnp.where(cid == 0, nch0, nch1)

        pltpu.sync_copy(z_hbm, acc.at[pl.ds(sid * rows_per_tile_z,
                                            rows_per_tile_z)])

        @pl.when(cid == 0)
        def _():
            pltpu.sync_copy(src0_hbm.at[sid], idx_s.at[pl.ds(0, nch0)])
            pltpu.sync_copy(dst0_hbm.at[sid], idx_d.at[pl.ds(0, nch0)])

        @pl.when(cid != 0)
        def _():
            pltpu.sync_copy(src1_hbm.at[sid], idx_s.at[pl.ds(0, nch1)])
            pltpu.sync_copy(dst1_hbm.at[sid], idx_d.at[pl.ds(0, nch1)])
        plsc.subcore_barrier()

        def chunk(c, carry):
            pltpu.async_copy(table_hbm.at[idx_s.at[c]], rows, sem).wait()
            pltpu.sync_copy(rows, acc.at[idx_d.at[c]], add=True)
            return carry

        lax.fori_loop(0, my_nch, chunk, 0, unroll=False)
        plsc.subcore_barrier()

        pltpu.sync_copy(
            acc.at[pl.ds(sid * rows_cp, rows_cp)],
            out_hbm.at[cid].at[pl.ds(sid * rows_cp, rows_cp)],
        )
        if tail:
            @pl.when(sid == ns - 1)
            def _():
                pltpu.sync_copy(
                    acc.at[pl.ds(rows_cp * ns, tail)],
                    out_hbm.at[cid].at[pl.ds(rows_cp * ns, tail)],
                )

    return body(table, src_r[0], dst_r[0], src_r[1], dst_r[1], zeros)


def _mlp_block(x_ref, p_ref, wa_ref, ca_ref, wb_ref, cb_ref):
    a = x_ref[...] + p_ref[0] + p_ref[1]
    t = jnp.dot(a, wa_ref[...], preferred_element_type=jnp.float32)
    t = jnp.maximum(t + ca_ref[...], 0.0)
    u = jnp.dot(t, wb_ref[...], preferred_element_type=jnp.float32)
    return u + cb_ref[...]


def _mlp1_body(x_ref, p_ref, wa_ref, ca_ref, wb_ref, cb_ref, o_ref):
    u = _mlp_block(x_ref, p_ref, wa_ref, ca_ref, wb_ref, cb_ref)
    o_ref[...] = jnp.maximum(u, 0.0)


def _mlp2_body(x_ref, p_ref, wa_ref, ca_ref, wb_ref, cb_ref, o_ref):
    u = _mlp_block(x_ref, p_ref, wa_ref, ca_ref, wb_ref, cb_ref)
    m = jnp.max(u, axis=1, keepdims=True)
    z = u - m
    lse = jnp.log(jnp.sum(jnp.exp(z), axis=1, keepdims=True))
    o_ref[...] = z - lse


def _tc_mlp(body_fn, x, p, wa, ca, wb, cb, *, block_rows=1000):
    n, d = x.shape
    grid = (n // block_rows,)
    return pl.pallas_call(
        body_fn,
        grid=grid,
        in_specs=[
            pl.BlockSpec((block_rows, d), lambda i: (i, 0)),
            pl.BlockSpec((p.shape[0], block_rows, d), lambda i: (0, i, 0)),
            pl.BlockSpec((d, d), lambda i: (0, 0)),
            pl.BlockSpec((1, d), lambda i: (0, 0)),
            pl.BlockSpec((d, d), lambda i: (0, 0)),
            pl.BlockSpec((1, d), lambda i: (0, 0)),
        ],
        out_specs=pl.BlockSpec((block_rows, d), lambda i: (i, 0)),
        out_shape=jax.ShapeDtypeStruct((n, d), jnp.float32),
    )(x, p, wa, ca, wb, cb)


def kernel(x, edge_index, W1, b1, g1, bt1, W2, b2, bn_g0, bn_b0,
           W3, b3, g2, bt2, W4, b4):
    info = plsc.get_sparse_core_info()
    nc, ns = info.num_cores, info.num_subcores
    nw = nc * ns

    inv = 1.0 / jnp.sqrt(1.0 + _BN_EPS)
    s1 = g1 * inv
    w1f = W1.T * s1[None, :]
    c1 = (b1 * s1 + bt1)[None, :]
    s0 = bn_g0 * inv
    w2f = W2.T * s0[None, :]
    c2 = (b2 * s0 + bn_b0)[None, :]
    s2 = g2 * inv
    w3f = W3.T * s2[None, :]
    c3 = (b3 * s2 + bt2)[None, :]
    w4f = W4.T
    c4 = b4[None, :]

    zr = (-(-(_N + 1) // ns) + 7) // 8 * 8
    acc_rows = zr * ns
    zeros = jnp.zeros((zr, _D), jnp.float32)

    e = edge_index.shape[1]
    nchunks = -(-e // _CH)
    nch0 = (nchunks * 49) // (80 * ns)
    nch1 = -(-(nchunks - nch0 * ns) // ns)
    cap = ns * (nch0 + nch1) * _CH
    npad = cap - e
    src = edge_index[0]
    dst = edge_index[1]
    pad_dst = _N + (jnp.arange(npad, dtype=jnp.int32) % (acc_rows - _N))
    src_p = jnp.concatenate([src, jnp.zeros((npad,), jnp.int32)])
    dst_p = jnp.concatenate([dst, pad_dst])

    def layout(arr):
        cut = ns * nch0 * _CH
        return (arr[:cut].reshape(ns, nch0, _CH),
                arr[cut:].reshape(ns, nch1, _CH))

    src_r = layout(src_p)
    dst_r = layout(dst_p)

    agg = functools.partial(_sc_agg, src_r=src_r, dst_r=dst_r, zeros=zeros,
                            nc=nc, ns=ns, nch0=nch0, nch1=nch1,
                            acc_rows=acc_rows)

    p1 = agg(x)
    h = _tc_mlp(_mlp1_body, x, p1, w1f, c1, w2f, c2)
    p2 = agg(h)
    return _tc_mlp(_mlp2_body, h, p2, w3f, c3, w4f, c4)
